# baseline (speedup 1.0000x reference)
import jax
import jax.numpy as jnp
from jax.experimental import pallas as pl
from jax.experimental.pallas import tpu as pltpu

N = 50000
F_IN = 128
SEQ_LEN = 12
H = 32
G = 4
NP = N // G
BP = 2560
HG = H * G
W4 = 4 * HG
CS = F_IN - 16
KA = HG + 64 + 16
KB = 2 * HG + 16


def _lstm_head_kernel(xr_ref, w0_ref, w1_ref, wf1_ref, bf1_ref, wf2_ref,
                      bf2_ref, y_ref, sa_ref, sb_ref):
    bf16 = jnp.bfloat16
    f32 = jnp.float32
    xr = xr_ref[...]

    xt = jnp.concatenate(
        [xr[:, g * F_IN + CS:g * F_IN + CS + 16] for g in range(G)],
        axis=1).astype(bf16)
    ones = jnp.ones((xt.shape[0], 16), bf16)
    zeros128 = jnp.zeros((xt.shape[0], HG), bf16)

    sa_ref[...] = jnp.concatenate([zeros128, xt, ones], axis=1)
    sb_ref[...] = jnp.concatenate([zeros128, zeros128, ones], axis=1)

    c0 = jnp.zeros((xt.shape[0], HG), f32)
    c1 = jnp.zeros((xt.shape[0], HG), f32)
    h1p = c1

    for j in range(SEQ_LEN):
        g = jnp.dot(sa_ref[...], w0_ref[:, j * W4:(j + 1) * W4],
                    preferred_element_type=f32)
        a = jnp.tanh(g)
        af, ai, ag, ao = (a[:, HG:2 * HG], a[:, 0:HG],
                          a[:, 2 * HG:3 * HG], a[:, 3 * HG:4 * HG])
        c0 = 0.5 * ((af * c0 + c0) + (ai * ag + ag))
        t = jnp.tanh(c0)
        h0p = ao * t + t
        h0b = h0p.astype(bf16)
        sa_ref[:, 0:HG] = h0b
        sb_ref[:, 0:HG] = h0b

        g1 = jnp.dot(sb_ref[...], w1_ref[...], preferred_element_type=f32)
        a1 = jnp.tanh(g1)
        af1, ai1, ag1, ao1 = (a1[:, HG:2 * HG], a1[:, 0:HG],
                              a1[:, 2 * HG:3 * HG], a1[:, 3 * HG:4 * HG])
        c1 = 0.5 * ((af1 * c1 + c1) + (ai1 * ag1 + ag1))
        t1 = jnp.tanh(c1)
        h1p = ao1 * t1 + t1
        sb_ref[:, HG:2 * HG] = h1p.astype(bf16)

    z = jax.nn.relu(
        jnp.dot(h1p.astype(bf16), wf1_ref[...],
                preferred_element_type=f32)
        + bf1_ref[...])
    y = jnp.dot(z.astype(bf16), wf2_ref[...], preferred_element_type=f32)
    y_ref[...] = y + bf2_ref[...]


def kernel(x, edge_index, W_fp, b_fp, W1, b1, W2, b2, W3, b3, W_tp, b_tp,
           W_ih0, W_hh0, b_ih0, b_hh0, W_ih1, W_hh1, b_ih1, b_hh1,
           W_f1, b_f1, W_f2, b_f2):
    f32 = jnp.float32
    bf16 = jnp.bfloat16
    xr = x.reshape(NP, G * F_IN)

    I4 = jnp.eye(G, dtype=f32)
    I12 = jnp.eye(SEQ_LEN, dtype=f32)
    half32 = jnp.full((H,), 0.5, f32)
    one32 = jnp.ones((H,), f32)
    sc = jnp.concatenate([half32, half32, one32, half32])[None, :]

    wh0s = (0.5 * W_hh0.T * sc).reshape(H, 4, H)
    wh0b = jnp.einsum('gh,kbu->gkbhu', I4, wh0s).reshape(HG, W4)
    wh0_tiled = jnp.tile(wh0b, (1, SEQ_LEN))

    v0 = ((W_tp @ W_ih0.T) * sc).reshape(4, H)
    wbd = jnp.einsum('jk,gh,bu->gjkbhu', I12, I4, v0).reshape(
        SEQ_LEN * G, SEQ_LEN * W4)
    wbd = wbd.reshape(G, SEQ_LEN, SEQ_LEN * W4)
    wbd = jnp.pad(wbd, ((0, 0), (4, 0), (0, 0))).reshape(
        G * 16, SEQ_LEN * W4)

    k0 = (((b_tp @ W_ih0.T + b_ih0 + b_hh0)[None, :]) * sc)
    k0big = jnp.broadcast_to(k0.reshape(4, 1, H), (4, G, H)).reshape(1, W4)
    k0rows = jnp.concatenate(
        [jnp.tile(k0big, (1, SEQ_LEN)), jnp.zeros((15, SEQ_LEN * W4), f32)],
        axis=0)
    w0all = jnp.concatenate([wh0_tiled, wbd, k0rows], axis=0)

    w1s = (0.5 * jnp.concatenate([W_ih1, W_hh1], axis=1).T * sc)
    w1a = w1s[0:H].reshape(H, 4, H)
    w1b = w1s[H:2 * H].reshape(H, 4, H)
    k1 = (((b_ih1 + b_hh1)[None, :]) * sc)
    k1big = jnp.broadcast_to(k1.reshape(4, 1, H), (4, G, H)).reshape(1, W4)
    w1aug = jnp.concatenate([
        jnp.einsum('gh,kbu->gkbhu', I4, w1a).reshape(HG, W4),
        jnp.einsum('gh,kbu->gkbhu', I4, w1b).reshape(HG, W4),
        k1big,
        jnp.zeros((15, W4), f32),
    ], axis=0)

    wf1b = jnp.einsum('gh,ku->gkhu', I4, 0.5 * W_f1).reshape(HG, 16 * G)
    bf1b = jnp.broadcast_to(b_f1[None, None, :], (1, G, 16)).reshape(1, 16 * G)
    wf2b = jnp.einsum('gh,u->guh', I4, W_f2[:, 0]).reshape(16 * G, G)
    bf2b = b_f2[None, :]

    w0all = w0all.astype(bf16)
    w1aug = w1aug.astype(bf16)
    wf1b = wf1b.astype(bf16)
    wf2b = wf2b.astype(bf16)

    full = lambda i: (0, 0)
    yp = pl.pallas_call(
        _lstm_head_kernel,
        grid=(pl.cdiv(NP, BP),),
        in_specs=[
            pl.BlockSpec((BP, G * F_IN), lambda i: (i, 0)),
            pl.BlockSpec(w0all.shape, full),
            pl.BlockSpec(w1aug.shape, full),
            pl.BlockSpec(wf1b.shape, full),
            pl.BlockSpec(bf1b.shape, full),
            pl.BlockSpec(wf2b.shape, full),
            pl.BlockSpec(bf2b.shape, full),
        ],
        out_specs=pl.BlockSpec((BP, G), lambda i: (i, 0)),
        out_shape=jax.ShapeDtypeStruct((NP, G), f32),
        scratch_shapes=[
            pltpu.VMEM((BP, KA), bf16),
            pltpu.VMEM((BP, KB), bf16),
        ],
    )(xr, w0all, w1aug, wf1b, bf1b, wf2b, bf2b)

    return yp.reshape(N, 1)


# speedup vs baseline: 1.2394x; 1.2394x over previous
import jax
import jax.numpy as jnp
from jax.experimental import pallas as pl
from jax.experimental.pallas import tpu as pltpu

N = 50000
F_IN = 128
SEQ_LEN = 12
H = 32
G = 4
NP = N // G
BP = 2560
HG = H * G
W4 = 4 * HG
CS = F_IN - 16
KA = HG + 64 + 16
KB = 2 * HG


def _lstm_head_kernel(xr_ref, w0_ref, w1_ref, k1_ref, wf1_ref, bf1_ref,
                      wf2_ref, bf2_ref, y_ref, sa_ref, sb_ref):
    bf16 = jnp.bfloat16
    f32 = jnp.float32
    xr = xr_ref[...]

    xt = jnp.concatenate(
        [xr[:, g * F_IN + CS:g * F_IN + CS + 16] for g in range(G)],
        axis=1).astype(bf16)
    ones = jnp.ones((xt.shape[0], 16), bf16)
    zeros128 = jnp.zeros((xt.shape[0], HG), bf16)

    sa_ref[...] = jnp.concatenate([zeros128, xt, ones], axis=1)
    sb_ref[...] = jnp.concatenate([zeros128, zeros128], axis=1)
    k1 = k1_ref[...]

    c0 = jnp.zeros((xt.shape[0], HG), f32)
    c1 = jnp.zeros((xt.shape[0], HG), f32)
    h1p = c1

    for j in range(SEQ_LEN):
        g = jnp.dot(sa_ref[...], w0_ref[:, j * W4:(j + 1) * W4],
                    preferred_element_type=f32)
        a = jnp.tanh(g)
        af, ai, ag, ao = (a[:, HG:2 * HG], a[:, 0:HG],
                          a[:, 2 * HG:3 * HG], a[:, 3 * HG:4 * HG])
        c0 = 0.5 * ((af * c0 + c0) + (ai * ag + ag))
        t = jnp.tanh(c0)
        h0p = ao * t + t
        h0b = h0p.astype(bf16)
        sa_ref[:, 0:HG] = h0b
        sb_ref[:, 0:HG] = h0b

        g1 = jnp.dot(sb_ref[...], w1_ref[...],
                     preferred_element_type=f32) + k1
        a1 = jnp.tanh(g1)
        af1, ai1, ag1, ao1 = (a1[:, HG:2 * HG], a1[:, 0:HG],
                              a1[:, 2 * HG:3 * HG], a1[:, 3 * HG:4 * HG])
        c1 = 0.5 * ((af1 * c1 + c1) + (ai1 * ag1 + ag1))
        t1 = jnp.tanh(c1)
        h1p = ao1 * t1 + t1
        sb_ref[:, HG:2 * HG] = h1p.astype(bf16)

    z = jax.nn.relu(
        jnp.dot(h1p.astype(bf16), wf1_ref[...],
                preferred_element_type=f32)
        + bf1_ref[...])
    y = jnp.dot(z.astype(bf16), wf2_ref[...], preferred_element_type=f32)
    y_ref[...] = y + bf2_ref[...]


def kernel(x, edge_index, W_fp, b_fp, W1, b1, W2, b2, W3, b3, W_tp, b_tp,
           W_ih0, W_hh0, b_ih0, b_hh0, W_ih1, W_hh1, b_ih1, b_hh1,
           W_f1, b_f1, W_f2, b_f2):
    f32 = jnp.float32
    bf16 = jnp.bfloat16
    xr = x.reshape(NP, G * F_IN)

    I4 = jnp.eye(G, dtype=f32)
    I12 = jnp.eye(SEQ_LEN, dtype=f32)
    half32 = jnp.full((H,), 0.5, f32)
    one32 = jnp.ones((H,), f32)
    sc = jnp.concatenate([half32, half32, one32, half32])[None, :]

    wh0s = (0.5 * W_hh0.T * sc).reshape(H, 4, H)
    wh0b = jnp.einsum('gh,kbu->gkbhu', I4, wh0s).reshape(HG, W4)
    wh0_tiled = jnp.tile(wh0b, (1, SEQ_LEN))

    v0 = ((W_tp @ W_ih0.T) * sc).reshape(4, H)
    wbd = jnp.einsum('jk,gh,bu->gjkbhu', I12, I4, v0).reshape(
        SEQ_LEN * G, SEQ_LEN * W4)
    wbd = wbd.reshape(G, SEQ_LEN, SEQ_LEN * W4)
    wbd = jnp.pad(wbd, ((0, 0), (4, 0), (0, 0))).reshape(
        G * 16, SEQ_LEN * W4)

    k0 = (((b_tp @ W_ih0.T + b_ih0 + b_hh0)[None, :]) * sc)
    k0big = jnp.broadcast_to(k0.reshape(4, 1, H), (4, G, H)).reshape(1, W4)
    k0rows = jnp.concatenate(
        [jnp.tile(k0big, (1, SEQ_LEN)), jnp.zeros((15, SEQ_LEN * W4), f32)],
        axis=0)
    w0all = jnp.concatenate([wh0_tiled, wbd, k0rows], axis=0)

    w1s = (0.5 * jnp.concatenate([W_ih1, W_hh1], axis=1).T * sc)
    w1a = w1s[0:H].reshape(H, 4, H)
    w1b = w1s[H:2 * H].reshape(H, 4, H)
    k1 = (((b_ih1 + b_hh1)[None, :]) * sc)
    k1big = jnp.broadcast_to(k1.reshape(4, 1, H), (4, G, H)).reshape(1, W4)
    w1aug = jnp.concatenate([
        jnp.einsum('gh,kbu->gkbhu', I4, w1a).reshape(HG, W4),
        jnp.einsum('gh,kbu->gkbhu', I4, w1b).reshape(HG, W4),
    ], axis=0)

    wf1b = jnp.einsum('gh,ku->gkhu', I4, 0.5 * W_f1).reshape(HG, 16 * G)
    bf1b = jnp.broadcast_to(b_f1[None, None, :], (1, G, 16)).reshape(1, 16 * G)
    wf2b = jnp.einsum('gh,u->guh', I4, W_f2[:, 0]).reshape(16 * G, G)
    bf2b = b_f2[None, :]

    w0all = w0all.astype(bf16)
    w1aug = w1aug.astype(bf16)
    wf1b = wf1b.astype(bf16)
    wf2b = wf2b.astype(bf16)

    full = lambda i: (0, 0)
    yp = pl.pallas_call(
        _lstm_head_kernel,
        grid=(pl.cdiv(NP, BP),),
        in_specs=[
            pl.BlockSpec((BP, G * F_IN), lambda i: (i, 0)),
            pl.BlockSpec(w0all.shape, full),
            pl.BlockSpec(w1aug.shape, full),
            pl.BlockSpec(k1big.shape, full),
            pl.BlockSpec(wf1b.shape, full),
            pl.BlockSpec(bf1b.shape, full),
            pl.BlockSpec(wf2b.shape, full),
            pl.BlockSpec(bf2b.shape, full),
        ],
        out_specs=pl.BlockSpec((BP, G), lambda i: (i, 0)),
        out_shape=jax.ShapeDtypeStruct((NP, G), f32),
        scratch_shapes=[
            pltpu.VMEM((BP, KA), bf16),
            pltpu.VMEM((BP, KB), bf16),
        ],
    )(xr, w0all, w1aug, k1big, wf1b, bf1b, wf2b, bf2b)

    return yp.reshape(N, 1)


# speedup vs baseline: 1.2417x; 1.0018x over previous
import jax
import jax.numpy as jnp
from jax.experimental import pallas as pl
from jax.experimental.pallas import tpu as pltpu

N = 50000
F_IN = 128
SEQ_LEN = 12
H = 32
G = 4
NP = N // G
BP = 1280
HG = H * G
W4 = 4 * HG
CS = F_IN - 16
KA = HG + 64 + 16
KB = 2 * HG


def _lstm_head_kernel(xr_ref, w0_ref, w1_ref, k1_ref, wf1_ref, bf1_ref,
                      wf2_ref, bf2_ref, y_ref, sa_ref, sb_ref):
    bf16 = jnp.bfloat16
    f32 = jnp.float32
    xr = xr_ref[...]

    xt = jnp.concatenate(
        [xr[:, g * F_IN + CS:g * F_IN + CS + 16] for g in range(G)],
        axis=1).astype(bf16)
    ones = jnp.ones((xt.shape[0], 16), bf16)
    zeros128 = jnp.zeros((xt.shape[0], HG), bf16)

    sa_ref[...] = jnp.concatenate([zeros128, xt, ones], axis=1)
    sb_ref[...] = jnp.concatenate([zeros128, zeros128], axis=1)
    k1 = k1_ref[...]

    c0 = jnp.zeros((xt.shape[0], HG), f32)
    c1 = jnp.zeros((xt.shape[0], HG), f32)
    h1p = c1

    for j in range(SEQ_LEN):
        g = jnp.dot(sa_ref[...], w0_ref[:, j * W4:(j + 1) * W4],
                    preferred_element_type=f32)
        a = jnp.tanh(g)
        af, ai, ag, ao = (a[:, HG:2 * HG], a[:, 0:HG],
                          a[:, 2 * HG:3 * HG], a[:, 3 * HG:4 * HG])
        c0 = 0.5 * ((af * c0 + c0) + (ai * ag + ag))
        t = jnp.tanh(c0)
        h0p = ao * t + t
        h0b = h0p.astype(bf16)
        sa_ref[:, 0:HG] = h0b
        sb_ref[:, 0:HG] = h0b

        g1 = jnp.dot(sb_ref[...], w1_ref[...],
                     preferred_element_type=f32) + k1
        a1 = jnp.tanh(g1)
        af1, ai1, ag1, ao1 = (a1[:, HG:2 * HG], a1[:, 0:HG],
                              a1[:, 2 * HG:3 * HG], a1[:, 3 * HG:4 * HG])
        c1 = 0.5 * ((af1 * c1 + c1) + (ai1 * ag1 + ag1))
        t1 = jnp.tanh(c1)
        h1p = ao1 * t1 + t1
        sb_ref[:, HG:2 * HG] = h1p.astype(bf16)

    z = jax.nn.relu(
        jnp.dot(h1p.astype(bf16), wf1_ref[...],
                preferred_element_type=f32)
        + bf1_ref[...])
    y = jnp.dot(z.astype(bf16), wf2_ref[...], preferred_element_type=f32)
    y_ref[...] = y + bf2_ref[...]


def kernel(x, edge_index, W_fp, b_fp, W1, b1, W2, b2, W3, b3, W_tp, b_tp,
           W_ih0, W_hh0, b_ih0, b_hh0, W_ih1, W_hh1, b_ih1, b_hh1,
           W_f1, b_f1, W_f2, b_f2):
    f32 = jnp.float32
    bf16 = jnp.bfloat16
    xr = x.reshape(NP, G * F_IN)

    I4 = jnp.eye(G, dtype=f32)
    I12 = jnp.eye(SEQ_LEN, dtype=f32)
    half32 = jnp.full((H,), 0.5, f32)
    one32 = jnp.ones((H,), f32)
    sc = jnp.concatenate([half32, half32, one32, half32])[None, :]

    wh0s = (0.5 * W_hh0.T * sc).reshape(H, 4, H)
    wh0b = jnp.einsum('gh,kbu->gkbhu', I4, wh0s).reshape(HG, W4)
    wh0_tiled = jnp.tile(wh0b, (1, SEQ_LEN))

    v0 = ((W_tp @ W_ih0.T) * sc).reshape(4, H)
    wbd = jnp.einsum('jk,gh,bu->gjkbhu', I12, I4, v0).reshape(
        SEQ_LEN * G, SEQ_LEN * W4)
    wbd = wbd.reshape(G, SEQ_LEN, SEQ_LEN * W4)
    wbd = jnp.pad(wbd, ((0, 0), (4, 0), (0, 0))).reshape(
        G * 16, SEQ_LEN * W4)

    k0 = (((b_tp @ W_ih0.T + b_ih0 + b_hh0)[None, :]) * sc)
    k0big = jnp.broadcast_to(k0.reshape(4, 1, H), (4, G, H)).reshape(1, W4)
    k0rows = jnp.concatenate(
        [jnp.tile(k0big, (1, SEQ_LEN)), jnp.zeros((15, SEQ_LEN * W4), f32)],
        axis=0)
    w0all = jnp.concatenate([wh0_tiled, wbd, k0rows], axis=0)

    w1s = (0.5 * jnp.concatenate([W_ih1, W_hh1], axis=1).T * sc)
    w1a = w1s[0:H].reshape(H, 4, H)
    w1b = w1s[H:2 * H].reshape(H, 4, H)
    k1 = (((b_ih1 + b_hh1)[None, :]) * sc)
    k1big = jnp.broadcast_to(k1.reshape(4, 1, H), (4, G, H)).reshape(1, W4)
    w1aug = jnp.concatenate([
        jnp.einsum('gh,kbu->gkbhu', I4, w1a).reshape(HG, W4),
        jnp.einsum('gh,kbu->gkbhu', I4, w1b).reshape(HG, W4),
    ], axis=0)

    wf1b = jnp.einsum('gh,ku->gkhu', I4, 0.5 * W_f1).reshape(HG, 16 * G)
    bf1b = jnp.broadcast_to(b_f1[None, None, :], (1, G, 16)).reshape(1, 16 * G)
    wf2b = jnp.einsum('gh,u->guh', I4, W_f2[:, 0]).reshape(16 * G, G)
    bf2b = b_f2[None, :]

    w0all = w0all.astype(bf16)
    w1aug = w1aug.astype(bf16)
    wf1b = wf1b.astype(bf16)
    wf2b = wf2b.astype(bf16)

    full = lambda i: (0, 0)
    yp = pl.pallas_call(
        _lstm_head_kernel,
        grid=(pl.cdiv(NP, BP),),
        in_specs=[
            pl.BlockSpec((BP, G * F_IN), lambda i: (i, 0)),
            pl.BlockSpec(w0all.shape, full),
            pl.BlockSpec(w1aug.shape, full),
            pl.BlockSpec(k1big.shape, full),
            pl.BlockSpec(wf1b.shape, full),
            pl.BlockSpec(bf1b.shape, full),
            pl.BlockSpec(wf2b.shape, full),
            pl.BlockSpec(bf2b.shape, full),
        ],
        out_specs=pl.BlockSpec((BP, G), lambda i: (i, 0)),
        out_shape=jax.ShapeDtypeStruct((NP, G), f32),
        scratch_shapes=[
            pltpu.VMEM((BP, KA), bf16),
            pltpu.VMEM((BP, KB), bf16),
        ],
    )(xr, w0all, w1aug, k1big, wf1b, bf1b, wf2b, bf2b)

    return yp.reshape(N, 1)


# speedup vs baseline: 1.2550x; 1.0107x over previous
import jax
import jax.numpy as jnp
from jax.experimental import pallas as pl
from jax.experimental.pallas import tpu as pltpu

N = 50000
F_IN = 128
SEQ_LEN = 12
H = 32
G = 4
NP = N // G
BP = 1280
HG = H * G
W4 = 4 * HG
CS = F_IN - 16
KA = HG + 64 + 16
KB = 2 * HG


def _lstm_head_kernel(xr_ref, w0_ref, w1_ref, k1_ref, wf1_ref, bf1_ref,
                      wf2_ref, bf2_ref, y_ref, sa_ref, sb_ref):
    bf16 = jnp.bfloat16
    f32 = jnp.float32
    xr = xr_ref[...]

    xt = jnp.concatenate(
        [xr[:, g * F_IN + CS:g * F_IN + CS + 16] for g in range(G)],
        axis=1).astype(bf16)
    ones = jnp.ones((xt.shape[0], 16), bf16)
    zeros128 = jnp.zeros((xt.shape[0], HG), bf16)

    sa_ref[...] = jnp.concatenate([zeros128, xt, ones], axis=1)
    sb_ref[...] = jnp.concatenate([zeros128, zeros128], axis=1)
    k1 = k1_ref[...]

    BH = BP // 2
    rows = (slice(0, BH), slice(BH, BP))
    zc = jnp.zeros((BH, HG), f32)
    c0 = [zc, zc]
    c1 = [zc, zc]
    h1p = [zc, zc]

    def cell(a, c):
        af, ai, ag, ao = (a[:, HG:2 * HG], a[:, 0:HG],
                          a[:, 2 * HG:3 * HG], a[:, 3 * HG:4 * HG])
        cn = 0.5 * ((af * c + c) + (ai * ag + ag))
        t = jnp.tanh(cn)
        return cn, ao * t + t

    for j in range(SEQ_LEN):
        w0j = w0_ref[:, j * W4:(j + 1) * W4]
        a = [None, None]
        for i in (0, 1):
            a[i] = jnp.tanh(jnp.dot(sa_ref[rows[i], :], w0j,
                                    preferred_element_type=f32))
        for i in (0, 1):
            c0[i], h0p = cell(a[i], c0[i])
            h0b = h0p.astype(bf16)
            sa_ref[rows[i], 0:HG] = h0b
            sb_ref[rows[i], 0:HG] = h0b
        a1 = [None, None]
        for i in (0, 1):
            a1[i] = jnp.tanh(jnp.dot(sb_ref[rows[i], :], w1_ref[...],
                                     preferred_element_type=f32) + k1)
        for i in (0, 1):
            c1[i], h1p[i] = cell(a1[i], c1[i])
            sb_ref[rows[i], HG:2 * HG] = h1p[i].astype(bf16)

    for i in (0, 1):
        z = jax.nn.relu(
            jnp.dot(h1p[i].astype(bf16), wf1_ref[...],
                    preferred_element_type=f32)
            + bf1_ref[...])
        y = jnp.dot(z.astype(bf16), wf2_ref[...], preferred_element_type=f32)
        y_ref[rows[i], :] = y + bf2_ref[...]


def kernel(x, edge_index, W_fp, b_fp, W1, b1, W2, b2, W3, b3, W_tp, b_tp,
           W_ih0, W_hh0, b_ih0, b_hh0, W_ih1, W_hh1, b_ih1, b_hh1,
           W_f1, b_f1, W_f2, b_f2):
    f32 = jnp.float32
    bf16 = jnp.bfloat16
    xr = x.reshape(NP, G * F_IN)

    I4 = jnp.eye(G, dtype=f32)
    I12 = jnp.eye(SEQ_LEN, dtype=f32)
    half32 = jnp.full((H,), 0.5, f32)
    one32 = jnp.ones((H,), f32)
    sc = jnp.concatenate([half32, half32, one32, half32])[None, :]

    wh0s = (0.5 * W_hh0.T * sc).reshape(H, 4, H)
    wh0b = jnp.einsum('gh,kbu->gkbhu', I4, wh0s).reshape(HG, W4)
    wh0_tiled = jnp.tile(wh0b, (1, SEQ_LEN))

    v0 = ((W_tp @ W_ih0.T) * sc).reshape(4, H)
    wbd = jnp.einsum('jk,gh,bu->gjkbhu', I12, I4, v0).reshape(
        SEQ_LEN * G, SEQ_LEN * W4)
    wbd = wbd.reshape(G, SEQ_LEN, SEQ_LEN * W4)
    wbd = jnp.pad(wbd, ((0, 0), (4, 0), (0, 0))).reshape(
        G * 16, SEQ_LEN * W4)

    k0 = (((b_tp @ W_ih0.T + b_ih0 + b_hh0)[None, :]) * sc)
    k0big = jnp.broadcast_to(k0.reshape(4, 1, H), (4, G, H)).reshape(1, W4)
    k0rows = jnp.concatenate(
        [jnp.tile(k0big, (1, SEQ_LEN)), jnp.zeros((15, SEQ_LEN * W4), f32)],
        axis=0)
    w0all = jnp.concatenate([wh0_tiled, wbd, k0rows], axis=0)

    w1s = (0.5 * jnp.concatenate([W_ih1, W_hh1], axis=1).T * sc)
    w1a = w1s[0:H].reshape(H, 4, H)
    w1b = w1s[H:2 * H].reshape(H, 4, H)
    k1 = (((b_ih1 + b_hh1)[None, :]) * sc)
    k1big = jnp.broadcast_to(k1.reshape(4, 1, H), (4, G, H)).reshape(1, W4)
    w1aug = jnp.concatenate([
        jnp.einsum('gh,kbu->gkbhu', I4, w1a).reshape(HG, W4),
        jnp.einsum('gh,kbu->gkbhu', I4, w1b).reshape(HG, W4),
    ], axis=0)

    wf1b = jnp.einsum('gh,ku->gkhu', I4, 0.5 * W_f1).reshape(HG, 16 * G)
    bf1b = jnp.broadcast_to(b_f1[None, None, :], (1, G, 16)).reshape(1, 16 * G)
    wf2b = jnp.einsum('gh,u->guh', I4, W_f2[:, 0]).reshape(16 * G, G)
    bf2b = b_f2[None, :]

    w0all = w0all.astype(bf16)
    w1aug = w1aug.astype(bf16)
    wf1b = wf1b.astype(bf16)
    wf2b = wf2b.astype(bf16)

    full = lambda i: (0, 0)
    yp = pl.pallas_call(
        _lstm_head_kernel,
        grid=(pl.cdiv(NP, BP),),
        in_specs=[
            pl.BlockSpec((BP, G * F_IN), lambda i: (i, 0)),
            pl.BlockSpec(w0all.shape, full),
            pl.BlockSpec(w1aug.shape, full),
            pl.BlockSpec(k1big.shape, full),
            pl.BlockSpec(wf1b.shape, full),
            pl.BlockSpec(bf1b.shape, full),
            pl.BlockSpec(wf2b.shape, full),
            pl.BlockSpec(bf2b.shape, full),
        ],
        out_specs=pl.BlockSpec((BP, G), lambda i: (i, 0)),
        out_shape=jax.ShapeDtypeStruct((NP, G), f32),
        scratch_shapes=[
            pltpu.VMEM((BP, KA), bf16),
            pltpu.VMEM((BP, KB), bf16),
        ],
    )(xr, w0all, w1aug, k1big, wf1b, bf1b, wf2b, bf2b)

    return yp.reshape(N, 1)


# speedup vs baseline: 1.2678x; 1.0103x over previous
import jax
import jax.numpy as jnp
from jax.experimental import pallas as pl
from jax.experimental.pallas import tpu as pltpu

N = 50000
F_IN = 128
SEQ_LEN = 12
H = 32
G = 4
NP = N // G
BP = 1280
HG = H * G
W4 = 4 * HG
CS = F_IN - 16
KA = HG + 64 + 16
KB = 2 * HG


def _lstm_head_kernel(xr_ref, w0_ref, w1_ref, k1_ref, wf1_ref, bf1_ref,
                      wf2_ref, bf2_ref, y_ref, sa_ref, sb_ref):
    bf16 = jnp.bfloat16
    f32 = jnp.float32
    xr = xr_ref[...]

    xt = jnp.concatenate(
        [xr[:, g * F_IN + CS:g * F_IN + CS + 16] for g in range(G)],
        axis=1).astype(bf16)
    ones = jnp.ones((xt.shape[0], 16), bf16)
    zeros128 = jnp.zeros((xt.shape[0], HG), bf16)

    sa_ref[...] = jnp.concatenate([zeros128, xt, ones], axis=1)
    sb_ref[...] = jnp.concatenate([zeros128, zeros128], axis=1)
    k1 = k1_ref[...]

    BH = BP // 2
    rows = (slice(0, BH), slice(BH, BP))
    zc = jnp.zeros((BH, HG), f32)
    c0 = [zc, zc]
    c1 = [zc, zc]
    h1p = [zc, zc]

    def cell(a, c):
        af, ai, ag, ao = (a[:, HG:2 * HG], a[:, 0:HG],
                          a[:, 2 * HG:3 * HG], a[:, 3 * HG:4 * HG])
        if c is None:
            cn = 0.5 * (a[:, 0:HG] * a[:, 2 * HG:3 * HG]
                        + a[:, 2 * HG:3 * HG])
        else:
            cn = 0.5 * ((af * c + c) + (ai * ag + ag))
        t = jnp.tanh(cn)
        return cn, ao * t + t

    c0 = [None, None]
    c1 = [None, None]
    for j in range(SEQ_LEN):
        last = j == SEQ_LEN - 1
        w0j = w0_ref[:, j * W4:(j + 1) * W4]
        a = [None, None]
        for i in (0, 1):
            if j == 0:
                d = jnp.dot(sa_ref[rows[i], HG:KA], w0j[HG:KA, :],
                            preferred_element_type=f32)
            else:
                d = jnp.dot(sa_ref[rows[i], :], w0j,
                            preferred_element_type=f32)
            a[i] = jnp.tanh(d)
        for i in (0, 1):
            c0[i], h0p = cell(a[i], c0[i])
            h0b = h0p.astype(bf16)
            if not last:
                sa_ref[rows[i], 0:HG] = h0b
            sb_ref[rows[i], 0:HG] = h0b
        a1 = [None, None]
        for i in (0, 1):
            if j == 0:
                d1 = jnp.dot(sb_ref[rows[i], 0:HG], w1_ref[0:HG, :],
                             preferred_element_type=f32)
            else:
                d1 = jnp.dot(sb_ref[rows[i], :], w1_ref[...],
                             preferred_element_type=f32)
            a1[i] = jnp.tanh(d1 + k1)
        for i in (0, 1):
            c1[i], h1p[i] = cell(a1[i], c1[i])
            if not last:
                sb_ref[rows[i], HG:2 * HG] = h1p[i].astype(bf16)

    for i in (0, 1):
        z = jax.nn.relu(
            jnp.dot(h1p[i].astype(bf16), wf1_ref[...],
                    preferred_element_type=f32)
            + bf1_ref[...])
        y = jnp.dot(z.astype(bf16), wf2_ref[...], preferred_element_type=f32)
        y_ref[rows[i], :] = y + bf2_ref[...]


def kernel(x, edge_index, W_fp, b_fp, W1, b1, W2, b2, W3, b3, W_tp, b_tp,
           W_ih0, W_hh0, b_ih0, b_hh0, W_ih1, W_hh1, b_ih1, b_hh1,
           W_f1, b_f1, W_f2, b_f2):
    f32 = jnp.float32
    bf16 = jnp.bfloat16
    xr = x.reshape(NP, G * F_IN)

    I4 = jnp.eye(G, dtype=f32)
    I12 = jnp.eye(SEQ_LEN, dtype=f32)
    half32 = jnp.full((H,), 0.5, f32)
    one32 = jnp.ones((H,), f32)
    sc = jnp.concatenate([half32, half32, one32, half32])[None, :]

    wh0s = (0.5 * W_hh0.T * sc).reshape(H, 4, H)
    wh0b = jnp.einsum('gh,kbu->gkbhu', I4, wh0s).reshape(HG, W4)
    wh0_tiled = jnp.tile(wh0b, (1, SEQ_LEN))

    v0 = ((W_tp @ W_ih0.T) * sc).reshape(4, H)
    wbd = jnp.einsum('jk,gh,bu->gjkbhu', I12, I4, v0).reshape(
        SEQ_LEN * G, SEQ_LEN * W4)
    wbd = wbd.reshape(G, SEQ_LEN, SEQ_LEN * W4)
    wbd = jnp.pad(wbd, ((0, 0), (4, 0), (0, 0))).reshape(
        G * 16, SEQ_LEN * W4)

    k0 = (((b_tp @ W_ih0.T + b_ih0 + b_hh0)[None, :]) * sc)
    k0big = jnp.broadcast_to(k0.reshape(4, 1, H), (4, G, H)).reshape(1, W4)
    k0rows = jnp.concatenate(
        [jnp.tile(k0big, (1, SEQ_LEN)), jnp.zeros((15, SEQ_LEN * W4), f32)],
        axis=0)
    w0all = jnp.concatenate([wh0_tiled, wbd, k0rows], axis=0)

    w1s = (0.5 * jnp.concatenate([W_ih1, W_hh1], axis=1).T * sc)
    w1a = w1s[0:H].reshape(H, 4, H)
    w1b = w1s[H:2 * H].reshape(H, 4, H)
    k1 = (((b_ih1 + b_hh1)[None, :]) * sc)
    k1big = jnp.broadcast_to(k1.reshape(4, 1, H), (4, G, H)).reshape(1, W4)
    w1aug = jnp.concatenate([
        jnp.einsum('gh,kbu->gkbhu', I4, w1a).reshape(HG, W4),
        jnp.einsum('gh,kbu->gkbhu', I4, w1b).reshape(HG, W4),
    ], axis=0)

    wf1b = jnp.einsum('gh,ku->gkhu', I4, 0.5 * W_f1).reshape(HG, 16 * G)
    bf1b = jnp.broadcast_to(b_f1[None, None, :], (1, G, 16)).reshape(1, 16 * G)
    wf2b = jnp.einsum('gh,u->guh', I4, W_f2[:, 0]).reshape(16 * G, G)
    bf2b = b_f2[None, :]

    w0all = w0all.astype(bf16)
    w1aug = w1aug.astype(bf16)
    wf1b = wf1b.astype(bf16)
    wf2b = wf2b.astype(bf16)

    full = lambda i: (0, 0)
    yp = pl.pallas_call(
        _lstm_head_kernel,
        grid=(pl.cdiv(NP, BP),),
        in_specs=[
            pl.BlockSpec((BP, G * F_IN), lambda i: (i, 0)),
            pl.BlockSpec(w0all.shape, full),
            pl.BlockSpec(w1aug.shape, full),
            pl.BlockSpec(k1big.shape, full),
            pl.BlockSpec(wf1b.shape, full),
            pl.BlockSpec(bf1b.shape, full),
            pl.BlockSpec(wf2b.shape, full),
            pl.BlockSpec(bf2b.shape, full),
        ],
        out_specs=pl.BlockSpec((BP, G), lambda i: (i, 0)),
        out_shape=jax.ShapeDtypeStruct((NP, G), f32),
        scratch_shapes=[
            pltpu.VMEM((BP, KA), bf16),
            pltpu.VMEM((BP, KB), bf16),
        ],
    )(xr, w0all, w1aug, k1big, wf1b, bf1b, wf2b, bf2b)

    return yp.reshape(N, 1)
